# Initial kernel scaffold; baseline (speedup 1.0000x reference)
import jax
import jax.numpy as jnp
from jax.experimental import pallas as pl


def kernel(batch, emb, W1, b1, W2, b2):
    raise NotImplementedError("write your pallas kernel here")


# speedup vs baseline: 1.1823x; 1.1823x over previous
import functools

import jax
import jax.numpy as jnp
from jax import lax
from jax.experimental import pallas as pl
from jax.experimental.pallas import tpu as pltpu
from jax.experimental.pallas import tpu_sc as plsc

NUM_ITEMS = 1000000
EMBED_DIM = 64
H1 = 128
H2 = 64
BATCH = 16384
HIST = 50
TOTAL = BATCH * HIST

NC = 2
NS = 16
NW = NC * NS
B_PER_W = TOTAL // NW
CHUNK = 800
N_CHUNKS = B_PER_W // CHUNK


def _sc_gather(table, idx):
    mesh = plsc.VectorSubcoreMesh(core_axis_name="c", subcore_axis_name="s")

    @functools.partial(
        pl.kernel,
        out_type=jax.ShapeDtypeStruct((TOTAL, EMBED_DIM), jnp.float32),
        mesh=mesh,
        scratch_types=[
            pltpu.VMEM((CHUNK,), jnp.int32),
            pltpu.VMEM((CHUNK, EMBED_DIM), jnp.float32),
            pltpu.SemaphoreType.DMA,
        ],
        compiler_params=pltpu.CompilerParams(use_tc_tiling_on_sc=False),
    )
    def gather_kernel(table_hbm, idx_hbm, out_hbm, idx_v, rows_v, sem):
        wid = lax.axis_index("s") * NC + lax.axis_index("c")
        base = wid * B_PER_W

        def body(g, carry):
            off = base + g * CHUNK
            pltpu.sync_copy(idx_hbm.at[pl.ds(off, CHUNK)], idx_v)
            pltpu.async_copy(table_hbm.at[idx_v], rows_v, sem).wait()
            pltpu.sync_copy(rows_v, out_hbm.at[pl.ds(off, CHUNK)])
            return carry

        lax.fori_loop(0, N_CHUNKS, body, 0)

    return gather_kernel(table, idx)


ROWS_BLK = 4096


def _mlp_body(x_ref, w1_ref, b1_ref, w2_ref, b2_ref, o_ref):
    h = jnp.maximum(
        jnp.dot(x_ref[...], w1_ref[...], preferred_element_type=jnp.float32)
        + b1_ref[...],
        0.0,
    )
    o_ref[...] = jnp.maximum(
        jnp.dot(h, w2_ref[...], preferred_element_type=jnp.float32)
        + b2_ref[...],
        0.0,
    )


def _tc_mlp(x, W1, b1, W2, b2):
    grid = (TOTAL // ROWS_BLK,)
    return pl.pallas_call(
        _mlp_body,
        grid=grid,
        in_specs=[
            pl.BlockSpec((ROWS_BLK, EMBED_DIM), lambda i: (i, 0)),
            pl.BlockSpec((EMBED_DIM, H1), lambda i: (0, 0)),
            pl.BlockSpec((1, H1), lambda i: (0, 0)),
            pl.BlockSpec((H1, H2), lambda i: (0, 0)),
            pl.BlockSpec((1, H2), lambda i: (0, 0)),
        ],
        out_specs=pl.BlockSpec((ROWS_BLK, H2), lambda i: (i, 0)),
        out_shape=jax.ShapeDtypeStruct((TOTAL, H2), jnp.float32),
        compiler_params=pltpu.CompilerParams(
            dimension_semantics=("arbitrary",),
        ),
    )(x, W1, b1.reshape(1, H1), W2, b2.reshape(1, H2))


def kernel(batch, emb, W1, b1, W2, b2):
    idx = batch.reshape(-1).astype(jnp.int32)
    gathered = _sc_gather(emb, idx)
    out = _tc_mlp(gathered, W1, b1, W2, b2)
    return out.reshape(BATCH, HIST, H2)


# speedup vs baseline: 1.3825x; 1.1694x over previous
import functools

import jax
import jax.numpy as jnp
from jax import lax
from jax.experimental import pallas as pl
from jax.experimental.pallas import tpu as pltpu
from jax.experimental.pallas import tpu_sc as plsc

NUM_ITEMS = 1000000
EMBED_DIM = 64
H1 = 128
H2 = 64
BATCH = 16384
HIST = 50
TOTAL = BATCH * HIST
TOTAL2 = TOTAL // 2

NC = 2
NS = 16
NW = NC * NS
B_PER_W = TOTAL // NW
CHUNK = 800
N_CHUNKS = B_PER_W // CHUNK


B2_PER_W = TOTAL2 // NW
CHUNK2 = CHUNK // 2


def _sc_gather(table, idx_e, idx_o):
    mesh = plsc.VectorSubcoreMesh(core_axis_name="c", subcore_axis_name="s")

    @functools.partial(
        pl.kernel,
        out_type=jax.ShapeDtypeStruct((TOTAL2, 2 * EMBED_DIM), jnp.float32),
        mesh=mesh,
        scratch_types=[
            pltpu.VMEM((CHUNK2,), jnp.int32),
            pltpu.VMEM((CHUNK2,), jnp.int32),
            pltpu.VMEM((CHUNK2, EMBED_DIM), jnp.float32),
            pltpu.VMEM((CHUNK2, EMBED_DIM), jnp.float32),
            pltpu.SemaphoreType.DMA,
        ],
        compiler_params=pltpu.CompilerParams(use_tc_tiling_on_sc=False),
    )
    def gather_kernel(table_hbm, idxe_hbm, idxo_hbm, out_hbm,
                      idxe_v, idxo_v, rows_e, rows_o, sem):
        wid = lax.axis_index("s") * NC + lax.axis_index("c")
        base = wid * B2_PER_W

        def body(g, carry):
            off = base + g * CHUNK2
            pltpu.sync_copy(idxe_hbm.at[pl.ds(off, CHUNK2)], idxe_v)
            pltpu.sync_copy(idxo_hbm.at[pl.ds(off, CHUNK2)], idxo_v)
            pltpu.async_copy(table_hbm.at[idxe_v], rows_e, sem).wait()
            pltpu.async_copy(table_hbm.at[idxo_v], rows_o, sem).wait()
            pltpu.sync_copy(
                rows_e, out_hbm.at[pl.ds(off, CHUNK2), pl.ds(0, EMBED_DIM)])
            pltpu.sync_copy(
                rows_o,
                out_hbm.at[pl.ds(off, CHUNK2), pl.ds(EMBED_DIM, EMBED_DIM)])
            return carry

        lax.fori_loop(0, N_CHUNKS, body, 0)

    return gather_kernel(table, idx_e, idx_o)


ROWS_BLK = 4096


def _mlp_body(x_ref, w1_ref, b1_ref, w2_ref, b2_ref, o_ref):
    h = jnp.maximum(
        jnp.dot(x_ref[...], w1_ref[...], preferred_element_type=jnp.float32)
        + b1_ref[...],
        0.0,
    )
    o_ref[...] = jnp.maximum(
        jnp.dot(h, w2_ref[...], preferred_element_type=jnp.float32)
        + b2_ref[...],
        0.0,
    )


def _tc_mlp(x2, W1, b1, W2, b2):
    z1 = jnp.zeros((EMBED_DIM, H1), jnp.float32)
    W1b = jnp.block([[W1, z1], [z1, W1]])
    b1b = jnp.concatenate([b1, b1]).reshape(1, 2 * H1)
    z2 = jnp.zeros((H1, H2), jnp.float32)
    W2b = jnp.block([[W2, z2], [z2, W2]])
    b2b = jnp.concatenate([b2, b2]).reshape(1, 2 * H2)

    grid = (TOTAL2 // ROWS_BLK,)
    return pl.pallas_call(
        _mlp_body,
        grid=grid,
        in_specs=[
            pl.BlockSpec((ROWS_BLK, 2 * EMBED_DIM), lambda i: (i, 0)),
            pl.BlockSpec((2 * EMBED_DIM, 2 * H1), lambda i: (0, 0)),
            pl.BlockSpec((1, 2 * H1), lambda i: (0, 0)),
            pl.BlockSpec((2 * H1, 2 * H2), lambda i: (0, 0)),
            pl.BlockSpec((1, 2 * H2), lambda i: (0, 0)),
        ],
        out_specs=pl.BlockSpec((ROWS_BLK, 2 * H2), lambda i: (i, 0)),
        out_shape=jax.ShapeDtypeStruct((TOTAL2, 2 * H2), jnp.float32),
        compiler_params=pltpu.CompilerParams(
            dimension_semantics=("arbitrary",),
        ),
    )(x2, W1b, b1b, W2b, b2b)


def kernel(batch, emb, W1, b1, W2, b2):
    idx2 = batch.reshape(TOTAL2, 2).astype(jnp.int32)
    gathered2 = _sc_gather(emb, idx2[:, 0], idx2[:, 1])
    out2 = _tc_mlp(gathered2, W1, b1, W2, b2)
    return out2.reshape(BATCH, HIST, H2)


# speedup vs baseline: 1.7553x; 1.2697x over previous
import functools

import jax
import jax.numpy as jnp
from jax import lax
from jax.experimental import pallas as pl
from jax.experimental.pallas import tpu as pltpu
from jax.experimental.pallas import tpu_sc as plsc

NUM_ITEMS = 1000000
EMBED_DIM = 64
H1 = 128
H2 = 64
BATCH = 16384
HIST = 50
TOTAL = BATCH * HIST

NC = 2
NS = 16
NW = NC * NS
B_PER_W = TOTAL // NW
CHUNK = 800
N_CHUNKS = B_PER_W // CHUNK

NBLK = 8192


def _mlp_t_body(xT_ref, w1T_ref, b1_ref, w2_ref, b2_ref, o_ref):
    xT = xT_ref[...].astype(jnp.bfloat16)
    w1T = w1T_ref[...].astype(jnp.bfloat16)
    hT = lax.dot_general(
        w1T, xT, (((1,), (0,)), ((), ())),
        preferred_element_type=jnp.float32,
    )
    hT = jnp.maximum(hT + b1_ref[...], 0.0).astype(jnp.bfloat16)
    w2 = w2_ref[...].astype(jnp.bfloat16)
    out = lax.dot_general(
        hT, w2, (((0,), (0,)), ((), ())),
        preferred_element_type=jnp.float32,
    )
    o_ref[...] = jnp.maximum(out + b2_ref[...], 0.0)


def _tc_mlp_table(embT, W1, b1, W2, b2):
    grid = (pl.cdiv(NUM_ITEMS, NBLK),)
    return pl.pallas_call(
        _mlp_t_body,
        grid=grid,
        in_specs=[
            pl.BlockSpec((EMBED_DIM, NBLK), lambda i: (0, i)),
            pl.BlockSpec((H1, EMBED_DIM), lambda i: (0, 0)),
            pl.BlockSpec((H1, 1), lambda i: (0, 0)),
            pl.BlockSpec((H1, H2), lambda i: (0, 0)),
            pl.BlockSpec((1, H2), lambda i: (0, 0)),
        ],
        out_specs=pl.BlockSpec((NBLK, H2), lambda i: (i, 0)),
        out_shape=jax.ShapeDtypeStruct((NUM_ITEMS, H2), jnp.float32),
        compiler_params=pltpu.CompilerParams(
            dimension_semantics=("arbitrary",),
        ),
    )(embT, W1.T, b1.reshape(H1, 1), W2, b2.reshape(1, H2))


def _sc_gather(table, idx):
    mesh = plsc.VectorSubcoreMesh(core_axis_name="c", subcore_axis_name="s")

    @functools.partial(
        pl.kernel,
        out_type=jax.ShapeDtypeStruct((TOTAL, H2), jnp.float32),
        mesh=mesh,
        scratch_types=[
            pltpu.VMEM((CHUNK,), jnp.int32),
            pltpu.VMEM((CHUNK, H2), jnp.float32),
            pltpu.SemaphoreType.DMA,
        ],
        compiler_params=pltpu.CompilerParams(use_tc_tiling_on_sc=False),
    )
    def gather_kernel(table_hbm, idx_hbm, out_hbm, idx_v, rows_v, sem):
        wid = lax.axis_index("s") * NC + lax.axis_index("c")
        base = wid * B_PER_W

        def body(g, carry):
            off = base + g * CHUNK
            pltpu.sync_copy(idx_hbm.at[pl.ds(off, CHUNK)], idx_v)
            pltpu.async_copy(table_hbm.at[idx_v], rows_v, sem).wait()
            pltpu.sync_copy(rows_v, out_hbm.at[pl.ds(off, CHUNK)])
            return carry

        lax.fori_loop(0, N_CHUNKS, body, 0)

    return gather_kernel(table, idx)


def kernel(batch, emb, W1, b1, W2, b2):
    idx = batch.reshape(-1).astype(jnp.int32)
    table_out = _tc_mlp_table(emb.T, W1, b1, W2, b2)
    out = _sc_gather(table_out, idx)
    return out.reshape(BATCH, HIST, H2)
